# baseline (speedup 1.0000x reference)
import functools

import jax
import jax.numpy as jnp
from jax import lax
from jax.experimental import pallas as pl
from jax.experimental.pallas import tpu as pltpu
from jax.experimental.pallas import tpu_sc as plsc

NC, NS = 2, 16
NW = NC * NS
B = 16384
D = 64
BPW = B // NW


def _gather_body(u_hbm, i_hbm, ut_hbm, it_hbm, ue_out, ie_out,
                 uidx, iidx, usem, isem):
    wid = lax.axis_index("s") * NC + lax.axis_index("c")
    base = wid * BPW
    pltpu.sync_copy(u_hbm.at[pl.ds(base, BPW)], uidx)
    pltpu.sync_copy(i_hbm.at[pl.ds(base, BPW)], iidx)

    def issue(jj, _):
        b = jj * 16
        vu = uidx[pl.ds(b, 16)]
        vi = iidx[pl.ds(b, 16)]
        for k in range(16):
            n = base + b + k
            pltpu.async_copy(ut_hbm.at[pl.ds(vu[k], 1)],
                             ue_out.at[pl.ds(n, 1)], usem)
            pltpu.async_copy(it_hbm.at[pl.ds(vi[k], 1)],
                             ie_out.at[pl.ds(n, 1)], isem)
        return _

    lax.fori_loop(0, BPW // 16, issue, 0)

    def drain(j, _):
        pltpu.make_async_copy(ut_hbm.at[pl.ds(0, 16)],
                              ue_out.at[pl.ds(base, 16)], usem).wait()
        pltpu.make_async_copy(it_hbm.at[pl.ds(0, 16)],
                              ie_out.at[pl.ds(base, 16)], isem).wait()
        return _

    lax.fori_loop(0, BPW // 16, drain, 0)


def _make_gather():
    mesh = plsc.VectorSubcoreMesh(core_axis_name="c", subcore_axis_name="s")
    return pl.kernel(
        _gather_body,
        out_type=(
            jax.ShapeDtypeStruct((B, D), jnp.float32),
            jax.ShapeDtypeStruct((B, D), jnp.float32),
        ),
        mesh=mesh,
        scratch_types=[
            pltpu.VMEM((BPW,), jnp.int32),
            pltpu.VMEM((BPW,), jnp.int32),
            pltpu.SemaphoreType.DMA,
            pltpu.SemaphoreType.DMA,
        ],
    )


BLK = 2048


def _mlp_body(ue, ie, w1a, w1b, b1, w2, b2, w3, b3, out):
    h = jnp.dot(ue[...], w1a[...], preferred_element_type=jnp.float32)
    h = h + jnp.dot(ie[...], w1b[...], preferred_element_type=jnp.float32)
    h = jnp.maximum(h + b1[...], 0.0)
    h = jnp.dot(h, w2[...], preferred_element_type=jnp.float32) + b2[...]
    h = jnp.maximum(h, 0.0)
    o = jnp.dot(h, w3[...], preferred_element_type=jnp.float32) + b3[...]
    out[...] = jax.nn.sigmoid(o)


def _mlp(ue, ie, W1a, W1b, b1, W2, b2, W3, b3):
    grid = (B // BLK,)
    full = lambda g: (0, 0)
    return pl.pallas_call(
        _mlp_body,
        grid=grid,
        in_specs=[
            pl.BlockSpec((BLK, D), lambda g: (g, 0)),
            pl.BlockSpec((BLK, D), lambda g: (g, 0)),
            pl.BlockSpec(W1a.shape, full),
            pl.BlockSpec(W1b.shape, full),
            pl.BlockSpec(b1.shape, full),
            pl.BlockSpec(W2.shape, full),
            pl.BlockSpec(b2.shape, full),
            pl.BlockSpec(W3.shape, full),
            pl.BlockSpec(b3.shape, full),
        ],
        out_specs=pl.BlockSpec((BLK, 1), lambda g: (g, 0)),
        out_shape=jax.ShapeDtypeStruct((B, 1), jnp.float32),
    )(ue, ie, W1a, W1b, b1, W2, b2, W3, b3)


def kernel(u, i, user_table, item_table, W1, b1, W2, b2, W3, b3):
    ue, ie = _make_gather()(u, i, user_table, item_table)
    out = _mlp(ue, ie,
               W1[:D], W1[D:],
               b1.reshape(1, -1),
               W2, b2.reshape(1, -1),
               W3, b3.reshape(1, -1))
    return jnp.squeeze(out, axis=-1)


# speedup vs baseline: 1.0564x; 1.0564x over previous
import functools

import jax
import jax.numpy as jnp
from jax import lax
from jax.experimental import pallas as pl
from jax.experimental.pallas import tpu as pltpu
from jax.experimental.pallas import tpu_sc as plsc

NC, NS = 2, 16
NW = NC * NS
B = 16384
D = 64
DW = 2 * D
BPW = B // NW
CH = 128
NCH = BPW // CH
PAIRS = 500000


def _gather_body(u_hbm, i_hbm, ut_hbm, it_hbm, ue_out, ie_out,
                 uidx, iidx, upair, ipair,
                 gu0, gu1, gu2, gi0, gi1, gi2, gsem, wsem):
    wid = lax.axis_index("s") * NC + lax.axis_index("c")
    base = wid * BPW
    pltpu.sync_copy(u_hbm.at[pl.ds(base, BPW)], uidx)
    pltpu.sync_copy(i_hbm.at[pl.ds(base, BPW)], iidx)

    def topairs(t, _):
        sl = pl.ds(t * 16, 16)
        upair[sl] = jax.lax.shift_right_logical(uidx[sl], 1)
        ipair[sl] = jax.lax.shift_right_logical(iidx[sl], 1)
        return _

    lax.fori_loop(0, BPW // 16, topairs, 0)

    gus = (gu0, gu1, gu2)
    gis = (gi0, gi1, gi2)

    def issue_gather(c, s):
        sl = pl.ds(c * CH, CH)
        pltpu.async_copy(ut_hbm.at[upair.at[sl]], gus[s], gsem)
        pltpu.async_copy(it_hbm.at[ipair.at[sl]], gis[s], gsem)

    def wait_gather(s):
        pltpu.make_async_copy(ut_hbm.at[upair.at[pl.ds(0, CH)]], gus[s], gsem).wait()
        pltpu.make_async_copy(it_hbm.at[ipair.at[pl.ds(0, CH)]], gis[s], gsem).wait()

    def issue_write(c, s):
        dst = pl.ds(base + c * CH, CH)
        pltpu.async_copy(gus[s], ue_out.at[dst], wsem)
        pltpu.async_copy(gis[s], ie_out.at[dst], wsem)

    def wait_write(s):
        pltpu.make_async_copy(gus[s], ue_out.at[pl.ds(base, CH)], wsem).wait()
        pltpu.make_async_copy(gis[s], ie_out.at[pl.ds(base, CH)], wsem).wait()

    issue_gather(0, 0)
    issue_gather(1, 1)
    issue_gather(2, 2)
    wait_gather(0)
    issue_write(0, 0)
    wait_gather(1)
    issue_write(1, 1)
    wait_gather(2)
    issue_write(2, 2)
    wait_write(0)
    issue_gather(3, 0)
    wait_gather(0)
    issue_write(3, 0)
    wait_write(1)
    wait_write(2)
    wait_write(0)


def _make_gather():
    mesh = plsc.VectorSubcoreMesh(core_axis_name="c", subcore_axis_name="s")
    return pl.kernel(
        _gather_body,
        out_type=(
            jax.ShapeDtypeStruct((B, DW), jnp.float32),
            jax.ShapeDtypeStruct((B, DW), jnp.float32),
        ),
        mesh=mesh,
        scratch_types=[
            pltpu.VMEM((BPW,), jnp.int32),
            pltpu.VMEM((BPW,), jnp.int32),
            pltpu.VMEM((BPW,), jnp.int32),
            pltpu.VMEM((BPW,), jnp.int32),
            pltpu.VMEM((CH, DW), jnp.float32),
            pltpu.VMEM((CH, DW), jnp.float32),
            pltpu.VMEM((CH, DW), jnp.float32),
            pltpu.VMEM((CH, DW), jnp.float32),
            pltpu.VMEM((CH, DW), jnp.float32),
            pltpu.VMEM((CH, DW), jnp.float32),
            pltpu.SemaphoreType.DMA,
            pltpu.SemaphoreType.DMA,
        ],
    )


BLK = 2048


def _mlp_body(ue2, ie2, um, im, w1a, w1b, b1, w2, b2, w3, b3, out):
    mu = (um[...] & 1) == 0
    mi = (im[...] & 1) == 0
    ue = jnp.where(mu, ue2[:, :D], ue2[:, D:])
    ie = jnp.where(mi, ie2[:, :D], ie2[:, D:])
    h = jnp.dot(ue, w1a[...], preferred_element_type=jnp.float32)
    h = h + jnp.dot(ie, w1b[...], preferred_element_type=jnp.float32)
    h = jnp.maximum(h + b1[...], 0.0)
    h = jnp.dot(h, w2[...], preferred_element_type=jnp.float32) + b2[...]
    h = jnp.maximum(h, 0.0)
    o = jnp.dot(h, w3[...], preferred_element_type=jnp.float32) + b3[...]
    out[...] = jax.nn.sigmoid(o)


def _mlp(ue2, ie2, um, im, W1a, W1b, b1, W2, b2, W3, b3):
    grid = (B // BLK,)
    full = lambda g: (0, 0)
    return pl.pallas_call(
        _mlp_body,
        grid=grid,
        in_specs=[
            pl.BlockSpec((BLK, DW), lambda g: (g, 0)),
            pl.BlockSpec((BLK, DW), lambda g: (g, 0)),
            pl.BlockSpec((BLK, 1), lambda g: (g, 0)),
            pl.BlockSpec((BLK, 1), lambda g: (g, 0)),
            pl.BlockSpec(W1a.shape, full),
            pl.BlockSpec(W1b.shape, full),
            pl.BlockSpec(b1.shape, full),
            pl.BlockSpec(W2.shape, full),
            pl.BlockSpec(b2.shape, full),
            pl.BlockSpec(W3.shape, full),
            pl.BlockSpec(b3.shape, full),
        ],
        out_specs=pl.BlockSpec((BLK, 1), lambda g: (g, 0)),
        out_shape=jax.ShapeDtypeStruct((B, 1), jnp.float32),
    )(ue2, ie2, um, im, W1a, W1b, b1, W2, b2, W3, b3)


def kernel(u, i, user_table, item_table, W1, b1, W2, b2, W3, b3):
    ut2 = user_table.reshape(PAIRS, DW)
    it2 = item_table.reshape(PAIRS, DW)
    ue2, ie2 = _make_gather()(u, i, ut2, it2)
    out = _mlp(ue2, ie2, u.reshape(-1, 1), i.reshape(-1, 1),
               W1[:D], W1[D:],
               b1.reshape(1, -1),
               W2, b2.reshape(1, -1),
               W3, b3.reshape(1, -1))
    return jnp.squeeze(out, axis=-1)
